# Initial kernel scaffold; baseline (speedup 1.0000x reference)
import jax
import jax.numpy as jnp
from jax.experimental import pallas as pl


def kernel(x, edge_index, W0, b0, W1, b1, W2, b2, W3, b3, g0, be0, g1, be1, g2, be2):
    raise NotImplementedError("write your pallas kernel here")


# speedup vs baseline: 16.9308x; 16.9308x over previous
import functools

import jax
import jax.numpy as jnp
from jax import lax
from jax.experimental import pallas as pl
from jax.experimental.pallas import tpu as pltpu
from jax.experimental.pallas import tpu_sc as plsc

N = 10000
E = 320000
D = 128
C = 7
EPS = 1e-5

NC = 2
NS = 16
CH = 128

NW = NC * NS
NCH = -(-E // CH)
CPT = 8 * (-(-NCH // (NW * 8)))
E_PAD = NW * CPT * CH
ROWS_PT = 5 * CH
N_PAD = NS * ROWS_PT
N_DUMMY = N_PAD - N

_mesh = plsc.VectorSubcoreMesh(core_axis_name="c", subcore_axis_name="s")


def _sc_scatter_body(h_hbm, src_hbm, dst_hbm, zrows_hbm, out_hbm,
                     src_v, dst_v, rows_v, acc_sh, sem):
    c = lax.axis_index("c")
    s = lax.axis_index("s")
    wid = c * NS + s
    r0 = s * ROWS_PT
    pltpu.sync_copy(zrows_hbm, rows_v)
    for k in range(ROWS_PT // CH):
        pltpu.sync_copy(rows_v, acc_sh.at[pl.ds(r0 + k * CH, CH)])
    pltpu.sync_copy(src_hbm.at[pl.ds(wid * CPT, CPT)], src_v)
    pltpu.sync_copy(dst_hbm.at[pl.ds(wid * CPT, CPT)], dst_v)
    plsc.subcore_barrier()

    def body(j, carry):
        pltpu.async_copy(h_hbm.at[src_v.at[j]], rows_v, sem).wait()
        pltpu.sync_copy(rows_v, acc_sh.at[dst_v.at[j]], add=True)
        return carry

    lax.fori_loop(0, CPT, body, 0)
    plsc.subcore_barrier()
    for k in range(ROWS_PT // CH):
        pltpu.sync_copy(acc_sh.at[pl.ds(r0 + k * CH, CH)],
                        out_hbm.at[c, pl.ds(r0 + k * CH, CH)])


_sc_scatter = functools.partial(
    pl.kernel,
    out_type=jax.ShapeDtypeStruct((NC, N_PAD, D), jnp.float32),
    mesh=_mesh,
    scratch_types=[
        pltpu.VMEM((CPT, CH), jnp.int32),
        pltpu.VMEM((CPT, CH), jnp.int32),
        pltpu.VMEM((CH, D), jnp.float32),
        pltpu.VMEM_SHARED((N_PAD, D), jnp.float32),
        pltpu.SemaphoreType.DMA,
    ],
)(_sc_scatter_body)


def _sc_degree_body(dst_hbm, ones_hbm, zrow_hbm, out_hbm,
                    dst_v, ones_v, zrow_v, acc_sh):
    c = lax.axis_index("c")
    s = lax.axis_index("s")
    wid = c * NS + s
    r0 = s * ROWS_PT
    pltpu.sync_copy(ones_hbm, ones_v)
    pltpu.sync_copy(zrow_hbm, zrow_v)
    for k in range(ROWS_PT // CH):
        pltpu.sync_copy(zrow_v, acc_sh.at[pl.ds(r0 + k * CH, CH)])
    pltpu.sync_copy(dst_hbm.at[pl.ds(wid * CPT, CPT)], dst_v)
    plsc.subcore_barrier()

    def body(j, carry):
        pltpu.sync_copy(ones_v, acc_sh.at[dst_v.at[j]], add=True)
        return carry

    lax.fori_loop(0, CPT, body, 0)
    plsc.subcore_barrier()
    for k in range(ROWS_PT // CH):
        pltpu.sync_copy(acc_sh.at[pl.ds(r0 + k * CH, CH)],
                        out_hbm.at[c, pl.ds(r0 + k * CH, CH)])


_sc_degree = functools.partial(
    pl.kernel,
    out_type=jax.ShapeDtypeStruct((NC, N_PAD), jnp.float32),
    mesh=_mesh,
    scratch_types=[
        pltpu.VMEM((CPT, CH), jnp.int32),
        pltpu.VMEM((CH,), jnp.float32),
        pltpu.VMEM((CH,), jnp.float32),
        pltpu.VMEM_SHARED((N_PAD,), jnp.float32),
    ],
)(_sc_degree_body)


def _b0_body(x_ref, w_ref, deg_ref, dinv_ref, hs_ref):
    deg = deg_ref[0] + deg_ref[1]
    dinv = lax.rsqrt(deg + 1.0)
    dinv_ref[...] = dinv
    hs_ref[...] = jnp.dot(x_ref[...], w_ref[...],
                          preferred_element_type=jnp.float32) * dinv


_tc_b0 = pl.pallas_call(
    _b0_body,
    out_shape=(jax.ShapeDtypeStruct((N, 1), jnp.float32),
               jax.ShapeDtypeStruct((N, D), jnp.float32)),
)


def _mk_layer(with_identity, with_matmul):
    def body(*refs):
        it = iter(refs)
        s_ref = next(it)
        hs_ref = next(it)
        dinv_ref = next(it)
        b_ref = next(it)
        g_ref = next(it)
        be_ref = next(it)
        w_ref = next(it) if with_matmul else None
        id_ref = next(it) if with_identity else None
        if with_matmul:
            h_out_ref = next(it)
        hsn_ref = next(it)

        dinv = dinv_ref[...]
        u = dinv * (s_ref[0, :N, :] + s_ref[1, :N, :] + hs_ref[...]) + b_ref[...]
        mu = jnp.mean(u, axis=0, keepdims=True)
        xc = u - mu
        var = jnp.mean(xc * xc, axis=0, keepdims=True)
        h = jnp.maximum(xc * lax.rsqrt(var + EPS) * g_ref[...] + be_ref[...], 0.0)
        if with_identity:
            h = h + id_ref[...]
        if with_matmul:
            h_out_ref[...] = h
            hsn_ref[...] = jnp.dot(h, w_ref[...],
                                   preferred_element_type=jnp.float32) * dinv
        else:
            hsn_ref[...] = h * dinv
    return body


_tc_layer0 = pl.pallas_call(
    _mk_layer(False, True),
    out_shape=(jax.ShapeDtypeStruct((N, D), jnp.float32),
               jax.ShapeDtypeStruct((N, D), jnp.float32)),
)
_tc_layer_mid = pl.pallas_call(
    _mk_layer(True, True),
    out_shape=(jax.ShapeDtypeStruct((N, D), jnp.float32),
               jax.ShapeDtypeStruct((N, D), jnp.float32)),
)
_tc_layer_last = pl.pallas_call(
    _mk_layer(True, False),
    out_shape=jax.ShapeDtypeStruct((N, D), jnp.float32),
)


def _final_body(s_ref, hs_ref, dinv_ref, w_ref, b_ref, o_ref):
    u = dinv_ref[...] * (s_ref[0, :N, :] + s_ref[1, :N, :] + hs_ref[...])
    logits = jnp.dot(u, w_ref[...], preferred_element_type=jnp.float32) + b_ref[...]
    m = jnp.max(logits, axis=1, keepdims=True)
    lse = jnp.log(jnp.sum(jnp.exp(logits - m), axis=1, keepdims=True)) + m
    o_ref[...] = logits - lse


_tc_final = pl.pallas_call(
    _final_body,
    out_shape=jax.ShapeDtypeStruct((N, C), jnp.float32),
)


def kernel(x, edge_index, W0, b0, W1, b1, W2, b2, W3, b3,
           g0, be0, g1, be1, g2, be2):
    npad = E_PAD - E
    pad_src = jnp.arange(npad, dtype=jnp.int32) % N
    pad_dst = N + jnp.arange(npad, dtype=jnp.int32) % N_DUMMY
    src_p = jnp.concatenate([edge_index[0], pad_src]).reshape(NW * CPT, CH)
    dst_p = jnp.concatenate([edge_index[1], pad_dst]).reshape(NW * CPT, CH)
    zrows = jnp.zeros((CH, D), jnp.float32)
    ones_row = jnp.ones((CH,), jnp.float32)
    zrow = jnp.zeros((CH,), jnp.float32)

    deg_p = _sc_degree(dst_p, ones_row, zrow)
    deg_in = deg_p[:, :N].reshape(NC, N, 1)

    dinv, hs0 = _tc_b0(x, W0, deg_in)
    s0 = _sc_scatter(hs0, src_p, dst_p, zrows)
    h1, hs1 = _tc_layer0(s0, hs0, dinv, b0, g0, be0, W1)
    s1 = _sc_scatter(hs1, src_p, dst_p, zrows)
    h2, hs2 = _tc_layer_mid(s1, hs1, dinv, b1, g1, be1, W2, h1)
    s2 = _sc_scatter(hs2, src_p, dst_p, zrows)
    hs3 = _tc_layer_last(s2, hs2, dinv, b2, g2, be2, h2)
    s3 = _sc_scatter(hs3, src_p, dst_p, zrows)
    return _tc_final(s3, hs3, dinv, W3, b3)


# speedup vs baseline: 24.7890x; 1.4641x over previous
import functools

import jax
import jax.numpy as jnp
from jax import lax
from jax.experimental import pallas as pl
from jax.experimental.pallas import tpu as pltpu
from jax.experimental.pallas import tpu_sc as plsc

N = 10000
E = 320000
D = 128
C = 7
EPS = 1e-5

NC = 2
NS = 16
CH = 128

NW = NC * NS
NCH = -(-E // CH)
CPT = 8 * (-(-NCH // (NW * 8)))
E_PAD = NW * CPT * CH
ROWS_PT = 5 * CH
N_PAD = NS * ROWS_PT
N_DUMMY = N_PAD - N

_mesh = plsc.VectorSubcoreMesh(core_axis_name="c", subcore_axis_name="s")


NBUF = 2
HALF = CPT // 2


def _sc_scatter_body(h_hbm, src_hbm, dst_hbm, zrows_hbm, out_hbm,
                     src_v, dst_v, rows_v, acc_sh, sem0, sem1):
    sems = (sem0, sem1)
    c = lax.axis_index("c")
    s = lax.axis_index("s")
    wid = c * NS + s
    r0 = s * ROWS_PT
    pltpu.sync_copy(zrows_hbm, rows_v.at[0])
    for k in range(ROWS_PT // CH):
        pltpu.sync_copy(rows_v.at[0], acc_sh.at[pl.ds(r0 + k * CH, CH)])
    plsc.subcore_barrier()

    for hh in range(CPT // HALF):
        base = wid * CPT + hh * HALF
        pltpu.sync_copy(src_hbm.at[pl.ds(base, HALF)], src_v)
        pltpu.sync_copy(dst_hbm.at[pl.ds(base, HALF)], dst_v)
        for b in range(NBUF - 1):
            pltpu.async_copy(h_hbm.at[src_v.at[b]], rows_v.at[b], sems[b])

        def body(g, carry):
            for b in range(NBUF):
                j = g * NBUF + b
                sb = (b + NBUF - 1) % NBUF

                @pl.when(j + NBUF - 1 < HALF)
                def _start():
                    pltpu.async_copy(h_hbm.at[src_v.at[j + NBUF - 1]],
                                     rows_v.at[sb], sems[sb])

                pltpu.make_async_copy(h_hbm.at[src_v.at[j]],
                                      rows_v.at[b], sems[b]).wait()
                pltpu.sync_copy(rows_v.at[b], acc_sh.at[dst_v.at[j]], add=True)
            return carry

        lax.fori_loop(0, HALF // NBUF, body, 0)
    plsc.subcore_barrier()
    for k in range(ROWS_PT // CH):
        pltpu.sync_copy(acc_sh.at[pl.ds(r0 + k * CH, CH)],
                        out_hbm.at[c, pl.ds(r0 + k * CH, CH)])


_sc_scatter = functools.partial(
    pl.kernel,
    out_type=jax.ShapeDtypeStruct((NC, N_PAD, D), jnp.float32),
    mesh=_mesh,
    scratch_types=[
        pltpu.VMEM((HALF, CH), jnp.int32),
        pltpu.VMEM((HALF, CH), jnp.int32),
        pltpu.VMEM((NBUF, CH, D), jnp.float32),
        pltpu.VMEM_SHARED((N_PAD, D), jnp.float32),
        pltpu.SemaphoreType.DMA,
        pltpu.SemaphoreType.DMA,
    ],
)(_sc_scatter_body)


def _sc_degree_body(dst_hbm, ones_hbm, zrow_hbm, out_hbm,
                    dst_v, ones_v, zrow_v, acc_sh):
    c = lax.axis_index("c")
    s = lax.axis_index("s")
    wid = c * NS + s
    r0 = s * ROWS_PT
    pltpu.sync_copy(ones_hbm, ones_v)
    pltpu.sync_copy(zrow_hbm, zrow_v)
    for k in range(ROWS_PT // CH):
        pltpu.sync_copy(zrow_v, acc_sh.at[pl.ds(r0 + k * CH, CH)])
    pltpu.sync_copy(dst_hbm.at[pl.ds(wid * CPT, CPT)], dst_v)
    plsc.subcore_barrier()

    def body(j, carry):
        pltpu.sync_copy(ones_v, acc_sh.at[dst_v.at[j]], add=True)
        return carry

    lax.fori_loop(0, CPT, body, 0)
    plsc.subcore_barrier()
    for k in range(ROWS_PT // CH):
        pltpu.sync_copy(acc_sh.at[pl.ds(r0 + k * CH, CH)],
                        out_hbm.at[c, pl.ds(r0 + k * CH, CH)])


_sc_degree = functools.partial(
    pl.kernel,
    out_type=jax.ShapeDtypeStruct((NC, N_PAD), jnp.float32),
    mesh=_mesh,
    scratch_types=[
        pltpu.VMEM((CPT, CH), jnp.int32),
        pltpu.VMEM((CH,), jnp.float32),
        pltpu.VMEM((CH,), jnp.float32),
        pltpu.VMEM_SHARED((N_PAD,), jnp.float32),
    ],
)(_sc_degree_body)


def _b0_body(x_ref, w_ref, deg_ref, dinv_ref, hs_ref):
    deg = deg_ref[0] + deg_ref[1]
    dinv = lax.rsqrt(deg + 1.0)
    dinv_ref[...] = dinv
    hs_ref[...] = jnp.dot(x_ref[...], w_ref[...],
                          preferred_element_type=jnp.float32) * dinv


_tc_b0 = pl.pallas_call(
    _b0_body,
    out_shape=(jax.ShapeDtypeStruct((N, 1), jnp.float32),
               jax.ShapeDtypeStruct((N, D), jnp.float32)),
)


def _mk_layer(with_identity, with_matmul):
    def body(*refs):
        it = iter(refs)
        s_ref = next(it)
        hs_ref = next(it)
        dinv_ref = next(it)
        b_ref = next(it)
        g_ref = next(it)
        be_ref = next(it)
        w_ref = next(it) if with_matmul else None
        id_ref = next(it) if with_identity else None
        if with_matmul:
            h_out_ref = next(it)
        hsn_ref = next(it)

        dinv = dinv_ref[...]
        u = dinv * (s_ref[0, :N, :] + s_ref[1, :N, :] + hs_ref[...]) + b_ref[...]
        mu = jnp.mean(u, axis=0, keepdims=True)
        xc = u - mu
        var = jnp.mean(xc * xc, axis=0, keepdims=True)
        h = jnp.maximum(xc * lax.rsqrt(var + EPS) * g_ref[...] + be_ref[...], 0.0)
        if with_identity:
            h = h + id_ref[...]
        if with_matmul:
            h_out_ref[...] = h
            hsn_ref[...] = jnp.dot(h, w_ref[...],
                                   preferred_element_type=jnp.float32) * dinv
        else:
            hsn_ref[...] = h * dinv
    return body


_tc_layer0 = pl.pallas_call(
    _mk_layer(False, True),
    out_shape=(jax.ShapeDtypeStruct((N, D), jnp.float32),
               jax.ShapeDtypeStruct((N, D), jnp.float32)),
)
_tc_layer_mid = pl.pallas_call(
    _mk_layer(True, True),
    out_shape=(jax.ShapeDtypeStruct((N, D), jnp.float32),
               jax.ShapeDtypeStruct((N, D), jnp.float32)),
)
_tc_layer_last = pl.pallas_call(
    _mk_layer(True, False),
    out_shape=jax.ShapeDtypeStruct((N, D), jnp.float32),
)


def _final_body(s_ref, hs_ref, dinv_ref, w_ref, b_ref, o_ref):
    u = dinv_ref[...] * (s_ref[0, :N, :] + s_ref[1, :N, :] + hs_ref[...])
    logits = jnp.dot(u, w_ref[...], preferred_element_type=jnp.float32) + b_ref[...]
    m = jnp.max(logits, axis=1, keepdims=True)
    lse = jnp.log(jnp.sum(jnp.exp(logits - m), axis=1, keepdims=True)) + m
    o_ref[...] = logits - lse


_tc_final = pl.pallas_call(
    _final_body,
    out_shape=jax.ShapeDtypeStruct((N, C), jnp.float32),
)


def kernel(x, edge_index, W0, b0, W1, b1, W2, b2, W3, b3,
           g0, be0, g1, be1, g2, be2):
    npad = E_PAD - E
    pad_src = jnp.arange(npad, dtype=jnp.int32) % N
    pad_dst = N + jnp.arange(npad, dtype=jnp.int32) % N_DUMMY
    src_p = jnp.concatenate([edge_index[0], pad_src]).reshape(NW * CPT, CH)
    dst_p = jnp.concatenate([edge_index[1], pad_dst]).reshape(NW * CPT, CH)
    zrows = jnp.zeros((CH, D), jnp.float32)
    ones_row = jnp.ones((CH,), jnp.float32)
    zrow = jnp.zeros((CH,), jnp.float32)

    deg_p = _sc_degree(dst_p, ones_row, zrow)
    deg_in = deg_p[:, :N].reshape(NC, N, 1)

    dinv, hs0 = _tc_b0(x, W0, deg_in)
    s0 = _sc_scatter(hs0, src_p, dst_p, zrows)
    h1, hs1 = _tc_layer0(s0, hs0, dinv, b0, g0, be0, W1)
    s1 = _sc_scatter(hs1, src_p, dst_p, zrows)
    h2, hs2 = _tc_layer_mid(s1, hs1, dinv, b1, g1, be1, W2, h1)
    s2 = _sc_scatter(hs2, src_p, dst_p, zrows)
    hs3 = _tc_layer_last(s2, hs2, dinv, b2, g2, be2, h2)
    s3 = _sc_scatter(hs3, src_p, dst_p, zrows)
    return _tc_final(s3, hs3, dinv, W3, b3)
